# Initial kernel scaffold; baseline (speedup 1.0000x reference)
import functools

import jax
import jax.numpy as jnp
from jax import lax
from jax.experimental import pallas as pl
from jax.experimental.pallas import tpu as pltpu
from jax.experimental.pallas import tpu_sc as plsc

VOCAB = 1000000
DIM = 64
BATCH = 16384
NEG = 20

NC = 2
NS = 16
NW = NC * NS

PB = BATCH // NW
CB = 32
NCHUNK = PB // CB
NROWS_NEG = CB * NEG
IDX_W = 128
NEG_GATHERS = NROWS_NEG // IDX_W
NEG_IDX_ROWS = PB * NEG // IDX_W


def _body(center_idx_h, pos_idx_h, neg_idx_h, center_table_h, context_table_h,
          pos_out_h, neg_out_h,
          cidx_v, pidx_v, nidx_v, crows_v, prows_v, nrows_v,
          out_pos_v, out_neg_v, sem):
    wid = lax.axis_index("s") * NC + lax.axis_index("c")
    base = wid * PB

    pltpu.sync_copy(center_idx_h.at[pl.ds(base, PB)], cidx_v)
    pltpu.sync_copy(pos_idx_h.at[pl.ds(base, PB)], pidx_v)
    pltpu.sync_copy(neg_idx_h.at[pl.ds(wid * NEG_IDX_ROWS, NEG_IDX_ROWS), :],
                    nidx_v)

    def chunk_body(c, _):
        cps = [
            pltpu.make_async_copy(
                center_table_h.at[cidx_v.at[pl.ds(c * CB, CB)]], crows_v, sem),
            pltpu.make_async_copy(
                context_table_h.at[pidx_v.at[pl.ds(c * CB, CB)]], prows_v, sem),
        ]
        for j in range(NEG_GATHERS):
            cps.append(pltpu.make_async_copy(
                context_table_h.at[nidx_v.at[c * NEG_GATHERS + j]],
                nrows_v.at[pl.ds(j * IDX_W, IDX_W), :], sem))
        for cp in cps:
            cp.start()
        for cp in cps:
            cp.wait()

        def elem_body(b, _):
            cvecs = [crows_v[b, pl.ds(q * 16, 16)] for q in range(4)]
            pvecs = [prows_v[b, pl.ds(q * 16, 16)] for q in range(4)]
            acc = cvecs[0] * pvecs[0]
            for q in range(1, 4):
                acc += cvecs[q] * pvecs[q]
            out_pos_v[c * CB + b] = jnp.sum(acc)
            for k in range(NEG):
                row = b * NEG + k
                acc = cvecs[0] * nrows_v[row, pl.ds(0, 16)]
                for q in range(1, 4):
                    acc += cvecs[q] * nrows_v[row, pl.ds(q * 16, 16)]
                out_neg_v[c * CB + b, k] = jnp.sum(acc)
            return 0

        lax.fori_loop(0, CB, elem_body, 0)
        return 0

    lax.fori_loop(0, NCHUNK, chunk_body, 0)

    pltpu.sync_copy(out_pos_v, pos_out_h.at[pl.ds(base, PB)])
    pltpu.sync_copy(out_neg_v, neg_out_h.at[pl.ds(base, PB), :])


@jax.jit
def _run(center_words, pos_words, neg_words, center_table, context_table):
    neg2d = neg_words.reshape(BATCH * NEG // IDX_W, IDX_W)
    mesh = plsc.VectorSubcoreMesh(core_axis_name="c", subcore_axis_name="s")
    f = pl.kernel(
        _body,
        out_type=(
            jax.ShapeDtypeStruct((BATCH,), jnp.float32),
            jax.ShapeDtypeStruct((BATCH, NEG), jnp.float32),
        ),
        mesh=mesh,
        scratch_types=[
            pltpu.VMEM((PB,), jnp.int32),
            pltpu.VMEM((PB,), jnp.int32),
            pltpu.VMEM((NEG_IDX_ROWS, IDX_W), jnp.int32),
            pltpu.VMEM((CB, DIM), jnp.float32),
            pltpu.VMEM((CB, DIM), jnp.float32),
            pltpu.VMEM((NROWS_NEG, DIM), jnp.float32),
            pltpu.VMEM((PB,), jnp.float32),
            pltpu.VMEM((PB, NEG), jnp.float32),
            pltpu.SemaphoreType.DMA,
        ],
    )
    return f(center_words, pos_words, neg2d, center_table, context_table)


def kernel(center_words, pos_words, neg_words, center_table, context_table):
    center_words = center_words.astype(jnp.int32)
    pos_words = pos_words.astype(jnp.int32)
    neg_words = neg_words.astype(jnp.int32)
    return _run(center_words, pos_words, neg_words, center_table,
                context_table)


# speedup vs baseline: 3.9806x; 3.9806x over previous
import functools

import jax
import jax.numpy as jnp
from jax import lax
from jax.experimental import pallas as pl
from jax.experimental.pallas import tpu as pltpu
from jax.experimental.pallas import tpu_sc as plsc

VOCAB = 1000000
DIM = 64
BATCH = 16384
NEG = 20

NC = 2
NS = 16
NW = NC * NS

PB = BATCH // NW
CB = 32
NCHUNK = PB // CB
NROWS_NEG = CB * NEG
IDX_W = 128
NEG_GATHERS = NROWS_NEG // IDX_W
NEG_IDX_ROWS = PB * NEG // IDX_W


def _body(center_idx_h, pos_idx_h, neg_idx_h, center_table_h, context_table_h,
          pos_out_h, neg_out_h,
          cidx_v, pidx_v, nidx_v, crows_v, prows_v, nrows_v,
          out_pos_v, out_neg_v, sem):
    wid = lax.axis_index("s") * NC + lax.axis_index("c")
    base = wid * PB

    pltpu.sync_copy(center_idx_h.at[pl.ds(base, PB)], cidx_v)
    pltpu.sync_copy(pos_idx_h.at[pl.ds(base, PB)], pidx_v)
    pltpu.sync_copy(neg_idx_h.at[pl.ds(wid * NEG_IDX_ROWS, NEG_IDX_ROWS), :],
                    nidx_v)

    def chunk_body(c, _):
        cps = [
            pltpu.make_async_copy(
                center_table_h.at[cidx_v.at[pl.ds(c * CB, CB)]], crows_v, sem),
            pltpu.make_async_copy(
                context_table_h.at[pidx_v.at[pl.ds(c * CB, CB)]], prows_v, sem),
        ]
        for j in range(NEG_GATHERS):
            cps.append(pltpu.make_async_copy(
                context_table_h.at[nidx_v.at[c * NEG_GATHERS + j]],
                nrows_v.at[pl.ds(j * IDX_W, IDX_W), :], sem))
        for cp in cps:
            cp.start()
        for cp in cps:
            cp.wait()

        lane = lax.iota(jnp.int32, 16)
        for g in range(CB // 16):
            rows = lane + g * 16
            rows_neg = rows * NEG

            def d_body(d, accs):
                col = jnp.full((16,), 0, jnp.int32) + d
                c_d = plsc.load_gather(crows_v, [rows, col])
                p_d = plsc.load_gather(prows_v, [rows, col])
                new = [accs[0] + c_d * p_d]
                for k in range(NEG):
                    n_d = plsc.load_gather(nrows_v, [rows_neg + k, col])
                    new.append(accs[1 + k] + c_d * n_d)
                return tuple(new)

            zero = jnp.zeros((16,), jnp.float32)
            accs = lax.fori_loop(0, DIM, d_body, (zero,) * (NEG + 1))
            base_e = c * CB + g * 16
            out_pos_v[pl.ds(base_e, 16)] = accs[0]
            flat = (base_e + lane) * NEG
            for k in range(NEG):
                plsc.store_scatter(out_neg_v, [flat + k], accs[1 + k])
        return 0

    lax.fori_loop(0, NCHUNK, chunk_body, 0)

    pltpu.sync_copy(out_pos_v, pos_out_h.at[pl.ds(base, PB)])
    pltpu.sync_copy(out_neg_v, neg_out_h.at[pl.ds(base * NEG, PB * NEG)])


@jax.jit
def _run(center_words, pos_words, neg_words, center_table, context_table):
    neg2d = neg_words.reshape(BATCH * NEG // IDX_W, IDX_W)
    mesh = plsc.VectorSubcoreMesh(core_axis_name="c", subcore_axis_name="s")
    f = pl.kernel(
        _body,
        out_type=(
            jax.ShapeDtypeStruct((BATCH,), jnp.float32),
            jax.ShapeDtypeStruct((BATCH * NEG,), jnp.float32),
        ),
        mesh=mesh,
        compiler_params=pltpu.CompilerParams(
            use_tc_tiling_on_sc=False, needs_layout_passes=False),
        scratch_types=[
            pltpu.VMEM((PB,), jnp.int32),
            pltpu.VMEM((PB,), jnp.int32),
            pltpu.VMEM((NEG_IDX_ROWS, IDX_W), jnp.int32),
            pltpu.VMEM((CB, DIM), jnp.float32),
            pltpu.VMEM((CB, DIM), jnp.float32),
            pltpu.VMEM((NROWS_NEG, DIM), jnp.float32),
            pltpu.VMEM((PB,), jnp.float32),
            pltpu.VMEM((PB * NEG,), jnp.float32),
            pltpu.SemaphoreType.DMA,
        ],
    )
    pos_probs, neg_flat = f(center_words, pos_words, neg2d, center_table,
                            context_table)
    return pos_probs, neg_flat.reshape(BATCH, NEG)


def kernel(center_words, pos_words, neg_words, center_table, context_table):
    center_words = center_words.astype(jnp.int32)
    pos_words = pos_words.astype(jnp.int32)
    neg_words = neg_words.astype(jnp.int32)
    return _run(center_words, pos_words, neg_words, center_table,
                context_table)


# speedup vs baseline: 4.0602x; 1.0200x over previous
import functools

import jax
import jax.numpy as jnp
from jax import lax
from jax.experimental import pallas as pl
from jax.experimental.pallas import tpu as pltpu
from jax.experimental.pallas import tpu_sc as plsc

VOCAB = 1000000
DIM = 64
BATCH = 16384
NEG = 20

NC = 2
NS = 16
NW = NC * NS

PB = BATCH // NW
CB = 32
NCHUNK = PB // CB
NROWS_NEG = CB * NEG
IDX_W = 128
NEG_GATHERS = NROWS_NEG // IDX_W
NEG_IDX_ROWS = PB * NEG // IDX_W


def _body(center_idx_h, pos_idx_h, neg_idx_h, center_table_h, context_table_h,
          pos_out_h, neg_out_h,
          cidx_v, pidx_v, nidx_v,
          crows0_v, prows0_v, nrows0_v, crows1_v, prows1_v, nrows1_v,
          out_pos_v, out_neg_v, sem0, sem1):
    wid = lax.axis_index("s") * NC + lax.axis_index("c")
    base = wid * PB

    pltpu.sync_copy(center_idx_h.at[pl.ds(base, PB)], cidx_v)
    pltpu.sync_copy(pos_idx_h.at[pl.ds(base, PB)], pidx_v)
    pltpu.sync_copy(neg_idx_h.at[pl.ds(wid * NEG_IDX_ROWS, NEG_IDX_ROWS), :],
                    nidx_v)

    bufs = ((crows0_v, prows0_v, nrows0_v, sem0),
            (crows1_v, prows1_v, nrows1_v, sem1))

    def descs(c, buf):
        crows_v, prows_v, nrows_v, sem = buf
        cps = [
            pltpu.make_async_copy(
                center_table_h.at[cidx_v.at[pl.ds(c * CB, CB)]], crows_v, sem),
            pltpu.make_async_copy(
                context_table_h.at[pidx_v.at[pl.ds(c * CB, CB)]], prows_v, sem),
        ]
        for j in range(NEG_GATHERS):
            cps.append(pltpu.make_async_copy(
                context_table_h.at[nidx_v.at[c * NEG_GATHERS + j]],
                nrows_v.at[pl.ds(j * IDX_W, IDX_W), :], sem))
        return cps

    def issue(c, buf):
        for cp in descs(c, buf):
            cp.start()

    def compute(c, buf):
        for cp in descs(c, buf):
            cp.wait()
        crows_v, prows_v, nrows_v, _ = buf
        lane = lax.iota(jnp.int32, 16)
        for g in range(CB // 16):
            rows = lane + g * 16
            rows_neg = rows * NEG

            def d_body(d, accs):
                col = jnp.full((16,), 0, jnp.int32) + d
                c_d = plsc.load_gather(crows_v, [rows, col])
                p_d = plsc.load_gather(prows_v, [rows, col])
                new = [accs[0] + c_d * p_d]
                for k in range(NEG):
                    n_d = plsc.load_gather(nrows_v, [rows_neg + k, col])
                    new.append(accs[1 + k] + c_d * n_d)
                return tuple(new)

            zero = jnp.zeros((16,), jnp.float32)
            accs = lax.fori_loop(0, DIM, d_body, (zero,) * (NEG + 1))
            base_e = c * CB + g * 16
            out_pos_v[pl.ds(base_e, 16)] = accs[0]
            flat = (base_e + lane) * NEG
            for k in range(NEG):
                plsc.store_scatter(out_neg_v, [flat + k], accs[1 + k])

    issue(0, bufs[0])

    def pair_body(i, _):
        c0 = 2 * i
        issue(c0 + 1, bufs[1])
        compute(c0, bufs[0])

        @pl.when(i < NCHUNK // 2 - 1)
        def _():
            issue(c0 + 2, bufs[0])

        compute(c0 + 1, bufs[1])
        return 0

    lax.fori_loop(0, NCHUNK // 2, pair_body, 0)

    pltpu.sync_copy(out_pos_v, pos_out_h.at[pl.ds(base, PB)])
    pltpu.sync_copy(out_neg_v, neg_out_h.at[pl.ds(base * NEG, PB * NEG)])


@jax.jit
def _run(center_words, pos_words, neg_words, center_table, context_table):
    neg2d = neg_words.reshape(BATCH * NEG // IDX_W, IDX_W)
    mesh = plsc.VectorSubcoreMesh(core_axis_name="c", subcore_axis_name="s")
    f = pl.kernel(
        _body,
        out_type=(
            jax.ShapeDtypeStruct((BATCH,), jnp.float32),
            jax.ShapeDtypeStruct((BATCH * NEG,), jnp.float32),
        ),
        mesh=mesh,
        compiler_params=pltpu.CompilerParams(
            use_tc_tiling_on_sc=False, needs_layout_passes=False),
        scratch_types=[
            pltpu.VMEM((PB,), jnp.int32),
            pltpu.VMEM((PB,), jnp.int32),
            pltpu.VMEM((NEG_IDX_ROWS, IDX_W), jnp.int32),
            pltpu.VMEM((CB, DIM), jnp.float32),
            pltpu.VMEM((CB, DIM), jnp.float32),
            pltpu.VMEM((NROWS_NEG, DIM), jnp.float32),
            pltpu.VMEM((CB, DIM), jnp.float32),
            pltpu.VMEM((CB, DIM), jnp.float32),
            pltpu.VMEM((NROWS_NEG, DIM), jnp.float32),
            pltpu.VMEM((PB,), jnp.float32),
            pltpu.VMEM((PB * NEG,), jnp.float32),
            pltpu.SemaphoreType.DMA,
            pltpu.SemaphoreType.DMA,
        ],
    )
    pos_probs, neg_flat = f(center_words, pos_words, neg2d, center_table,
                            context_table)
    return pos_probs, neg_flat.reshape(BATCH, NEG)


def kernel(center_words, pos_words, neg_words, center_table, context_table):
    center_words = center_words.astype(jnp.int32)
    pos_words = pos_words.astype(jnp.int32)
    neg_words = neg_words.astype(jnp.int32)
    return _run(center_words, pos_words, neg_words, center_table,
                context_table)
